# baseline (speedup 1.0000x reference)
import functools

import jax
import jax.numpy as jnp
from jax import lax
from jax.experimental import pallas as pl
from jax.experimental.pallas import tpu as pltpu
from jax.experimental.pallas import tpu_sc as plsc

N = 10000
D = 256
H = 128
NBASE = 2
NREL = 4
NE = 160000
EPS = 1e-5

CHUNK = 128
NCH = 160
GCH = 16
GCHA = 32
NSUB = 16
EP = NSUB * NCH * CHUNK
ACC_ROWS = 10240
DUMMY = N
CNT_ROWS = 40960
CNT_DUMMY = NREL * N


def _ln(t, g, b):
    mu = jnp.mean(t, axis=-1, keepdims=True)
    var = jnp.mean((t - mu) ** 2, axis=-1, keepdims=True)
    return (t - mu) * lax.rsqrt(var + EPS) * g + b


def _erf(z):
    a1, a2, a3, a4, a5 = (0.254829592, -0.284496736, 1.421413741,
                          -1.453152027, 1.061405429)
    p = 0.3275911
    az = jnp.abs(z)
    t = 1.0 / (1.0 + p * az)
    poly = ((((a5 * t + a4) * t + a3) * t + a2) * t + a1) * t
    e = 1.0 - poly * jnp.exp(-az * az)
    return jnp.sign(z) * e


def _gelu(u):
    return 0.5 * u * (1.0 + _erf(u * 0.7071067811865476))


def _dot(a, b):
    return lax.dot_general(a, b, (((1,), (0,)), ((), ())),
                           preferred_element_type=jnp.float32)


def _tc_base(nf, nt2, emb, g, b):
    def body(nf_ref, nt_ref, emb_ref, g_ref, b_ref, o_ref):
        nt = nt_ref[...]
        e = jnp.where(nt == 0, emb_ref[0:1, :],
                      jnp.where(nt == 1, emb_ref[1:2, :], emb_ref[2:3, :]))
        o_ref[...] = _ln(nf_ref[...] + e, g_ref[...], b_ref[...])

    R = 1000
    return pl.pallas_call(
        body,
        grid=(N // R,),
        in_specs=[
            pl.BlockSpec((R, D), lambda i: (i, 0)),
            pl.BlockSpec((R, 1), lambda i: (i, 0)),
            pl.BlockSpec((3, D), lambda i: (0, 0)),
            pl.BlockSpec((1, D), lambda i: (0, 0)),
            pl.BlockSpec((1, D), lambda i: (0, 0)),
        ],
        out_specs=pl.BlockSpec((R, D), lambda i: (i, 0)),
        out_shape=jax.ShapeDtypeStruct((N, D), jnp.float32),
    )(nf, nt2, emb, g, b)


def _tc_project(x, rel_w):
    def body(x_ref, w_ref, lo_ref, hi_ref):
        z = _dot(x_ref[...], w_ref[0])
        lo_ref[...] = z[:, :H]
        hi_ref[...] = z[:, H:]

    R = 1000
    nrb = N // R
    return pl.pallas_call(
        body,
        grid=(NREL, nrb),
        in_specs=[
            pl.BlockSpec((R, D), lambda r, i: (i, 0)),
            pl.BlockSpec((1, D, D), lambda r, i: (r, 0, 0)),
        ],
        out_specs=[
            pl.BlockSpec((R, H), lambda r, i: (r * nrb + i, 0)),
            pl.BlockSpec((R, H), lambda r, i: (r * nrb + i, 0)),
        ],
        out_shape=[
            jax.ShapeDtypeStruct((NREL * N, H), jnp.float32),
            jax.ShapeDtypeStruct((NREL * N, H), jnp.float32),
        ],
    )(x, rel_w)


def _tc_post(x, acc_lo, acc_hi, cnt_ta, cnt_tb, ball, sw, sb, n1g, n1b,
             w1, b1, w2, b2, n2g, n2b):
    def body(x_ref, lo_ref, hi_ref, cnta_ref, cntb_ref, ball_ref, sw_ref,
             sb_ref, n1g_ref, n1b_ref, w1_ref, b1_ref, w2_ref, b2_ref,
             n2g_ref, n2b_ref, o_ref):
        cnt = cnta_ref[...] + cntb_ref[...]
        deg = jnp.sum(cnt, axis=1, keepdims=True)
        bias_term = _dot(cnt, ball_ref[...])
        agg_pre = jnp.concatenate([lo_ref[...], hi_ref[...]], axis=1)
        agg = (agg_pre + bias_term) / jnp.maximum(deg, 1.0)
        xv = x_ref[...]
        h = _dot(xv, sw_ref[...]) + sb_ref[...] + agg
        x1 = _ln(xv + h, n1g_ref[...], n1b_ref[...])
        u = _gelu(_dot(x1, w1_ref[...]) + b1_ref[...])
        ff = _dot(u, w2_ref[...]) + b2_ref[...]
        o_ref[...] = _ln(x1 + ff, n2g_ref[...], n2b_ref[...])

    R = 1000
    return pl.pallas_call(
        body,
        grid=(N // R,),
        in_specs=[
            pl.BlockSpec((R, D), lambda i: (i, 0)),
            pl.BlockSpec((R, H), lambda i: (i, 0)),
            pl.BlockSpec((R, H), lambda i: (i, 0)),
            pl.BlockSpec((R, NREL), lambda i: (i, 0)),
            pl.BlockSpec((R, NREL), lambda i: (i, 0)),
            pl.BlockSpec((NREL, D), lambda i: (0, 0)),
            pl.BlockSpec((D, D), lambda i: (0, 0)),
            pl.BlockSpec((1, D), lambda i: (0, 0)),
            pl.BlockSpec((1, D), lambda i: (0, 0)),
            pl.BlockSpec((1, D), lambda i: (0, 0)),
            pl.BlockSpec((D, 2 * D), lambda i: (0, 0)),
            pl.BlockSpec((1, 2 * D), lambda i: (0, 0)),
            pl.BlockSpec((2 * D, D), lambda i: (0, 0)),
            pl.BlockSpec((1, D), lambda i: (0, 0)),
            pl.BlockSpec((1, D), lambda i: (0, 0)),
            pl.BlockSpec((1, D), lambda i: (0, 0)),
        ],
        out_specs=pl.BlockSpec((R, D), lambda i: (i, 0)),
        out_shape=jax.ShapeDtypeStruct((N, D), jnp.float32),
    )(x, acc_lo, acc_hi, cnt_ta, cnt_tb, ball, sw, sb, n1g, n1b,
      w1, b1, w2, b2, n2g, n2b)


def _tc_final(x, base, og, ob, gmv, dmv):
    def body(x_ref, base_ref, og_ref, ob_ref, gm_ref, dm_ref,
             o_ref, pool_ref):
        xln = _ln(x_ref[...], og_ref[...], ob_ref[...])
        bv = base_ref[...]
        xf = bv + gm_ref[...] * (xln - bv)
        o_ref[...] = xf
        i = pl.program_id(0)

        @pl.when(i == 0)
        def _():
            pool_ref[...] = jnp.zeros_like(pool_ref)

        dm = dm_ref[...]
        pool_ref[...] += jnp.sum(xf, axis=0, keepdims=True) * ((1.0 - dm) / N)

        @pl.when(i == 0)
        def _():
            pool_ref[...] += dm * xf[0:1, :]

    R = 1000
    return pl.pallas_call(
        body,
        grid=(N // R,),
        in_specs=[
            pl.BlockSpec((R, D), lambda i: (i, 0)),
            pl.BlockSpec((R, D), lambda i: (i, 0)),
            pl.BlockSpec((1, D), lambda i: (0, 0)),
            pl.BlockSpec((1, D), lambda i: (0, 0)),
            pl.BlockSpec((1, D), lambda i: (0, 0)),
            pl.BlockSpec((1, D), lambda i: (0, 0)),
        ],
        out_specs=[
            pl.BlockSpec((R, D), lambda i: (i, 0)),
            pl.BlockSpec((1, D), lambda i: (0, 0)),
        ],
        out_shape=[
            jax.ShapeDtypeStruct((N, D), jnp.float32),
            jax.ShapeDtypeStruct((1, D), jnp.float32),
        ],
    )(x, base, og, ob, gmv, dmv)


CNT_VROWS = CNT_ROWS // 8


SPREAD = 1024


def _sc_counts(clo_p, chi_p, onesrel, zeros128):
    mesh = plsc.VectorSubcoreMesh(core_axis_name="c", subcore_axis_name="s")

    @functools.partial(
        pl.kernel,
        mesh=mesh,
        out_type=jax.ShapeDtypeStruct((2, CNT_VROWS, CHUNK), jnp.float32),
        scratch_types=[
            pltpu.VMEM((GCH, CHUNK), jnp.int32),
            pltpu.VMEM((GCH, CHUNK), jnp.int32),
            pltpu.VMEM((2, CHUNK, CHUNK), jnp.float32),
            pltpu.VMEM_SHARED((CNT_VROWS, CHUNK), jnp.float32),
            pltpu.SemaphoreType.DMA((2,)),
            pltpu.SemaphoreType.DMA,
        ],
    )
    def k(clo_hbm, chi_hbm, ones_hbm, zer_hbm, out_hbm,
          lo_v, hi_v, ring_v, cnt_sp, gsem, sem):
        c = lax.axis_index("c")
        s = lax.axis_index("s")
        w = c * NSUB + s
        pltpu.sync_copy(zer_hbm, ring_v.at[0])
        rows = CNT_VROWS // NSUB
        zb = s * rows
        for kk in range(2):
            pltpu.sync_copy(ring_v.at[0],
                            cnt_sp.at[pl.ds(zb + kk * CHUNK, CHUNK)])
        pltpu.sync_copy(ring_v.at[0].at[pl.ds(0, 64)],
                        cnt_sp.at[pl.ds(zb + 2 * CHUNK, 64)])
        plsc.subcore_barrier()

        def gst(t, b):
            pltpu.make_async_copy(ones_hbm.at[lo_v.at[t]], ring_v.at[b],
                                  gsem.at[b]).start()

        def gwt(t, b):
            pltpu.make_async_copy(ones_hbm.at[lo_v.at[t]], ring_v.at[b],
                                  gsem.at[b]).wait()

        def sct(t, b):
            pltpu.sync_copy(ring_v.at[b], cnt_sp.at[hi_v.at[t]], add=True)

        @pl.loop(0, NCH // 2, step=GCH)
        def _(j0):
            pltpu.async_copy(clo_hbm.at[w, pl.ds(j0, GCH)], lo_v, sem).wait()
            pltpu.async_copy(chi_hbm.at[w, pl.ds(j0, GCH)], hi_v, sem).wait()
            gst(0, 0)
            gst(1, 1)

            @pl.loop(0, GCH - 2, step=2)
            def _(t0):
                for b in range(2):
                    t = t0 + b
                    gwt(t, b)
                    sct(t, b)
                    gst(t + 2, b)

            for b in range(2):
                t = GCH - 2 + b
                gwt(t, b)
                sct(t, b)

        plsc.subcore_barrier()

        @pl.when(c == 0)
        def _():
            pltpu.sync_copy(cnt_sp.at[pl.ds(zb, rows)],
                            out_hbm.at[0].at[pl.ds(zb, rows)])

        @pl.when(c == 1)
        def _():
            pltpu.sync_copy(cnt_sp.at[pl.ds(zb, rows)],
                            out_hbm.at[1].at[pl.ds(zb, rows)])

    return k(clo_p, chi_p, onesrel, zeros128)


def _sc_agg(y_lo, y_hi, gidx_p, sidx_p, zeros128):
    mesh = plsc.VectorSubcoreMesh(core_axis_name="c", subcore_axis_name="s")

    @functools.partial(
        pl.kernel,
        mesh=mesh,
        out_type=jax.ShapeDtypeStruct((2, ACC_ROWS, H), jnp.float32),
        scratch_types=[
            pltpu.VMEM((GCHA, CHUNK), jnp.int32),
            pltpu.VMEM((GCHA, CHUNK), jnp.int32),
            pltpu.VMEM((2, CHUNK, H), jnp.float32),
            pltpu.VMEM_SHARED((ACC_ROWS, H), jnp.float32),
            pltpu.SemaphoreType.DMA((2,)),
            pltpu.SemaphoreType.DMA((2,)),
            pltpu.SemaphoreType.DMA,
        ],
    )
    def k(ylo_hbm, yhi_hbm, g_hbm, s_hbm, z_hbm, out_hbm,
          g_v, s_v, ring_v, acc_sp, gsem, ssem, sem):
        c = lax.axis_index("c")
        s = lax.axis_index("s")
        pltpu.sync_copy(z_hbm, ring_v.at[0])
        zb = s * (ACC_ROWS // NSUB)
        for kk in range(ACC_ROWS // NSUB // CHUNK):
            pltpu.sync_copy(ring_v.at[0], acc_sp.at[pl.ds(zb + kk * CHUNK,
                                                          CHUNK)])
        plsc.subcore_barrier()

        def run(y_hbm, out_slice):
            def gst(t, b):
                pltpu.make_async_copy(y_hbm.at[g_v.at[t]], ring_v.at[b],
                                      gsem.at[b]).start()

            def gwt(t, b):
                pltpu.make_async_copy(y_hbm.at[g_v.at[t]], ring_v.at[b],
                                      gsem.at[b]).wait()

            def sst(t, b):
                pltpu.async_copy(ring_v.at[b], acc_sp.at[s_v.at[t]],
                                 ssem.at[b], add=True)

            def swt(t, b):
                pltpu.make_async_copy(ring_v.at[b], acc_sp.at[s_v.at[t]],
                                      ssem.at[b]).wait()

            @pl.loop(0, NCH, step=GCHA)
            def _(j0):
                pltpu.async_copy(g_hbm.at[s, pl.ds(j0, GCHA)], g_v,
                                 sem).wait()
                pltpu.async_copy(s_hbm.at[s, pl.ds(j0, GCHA)], s_v,
                                 sem).wait()
                gst(0, 0)
                gst(1, 1)

                @pl.loop(0, GCHA - 2, step=2)
                def _(t0):
                    for b in range(2):
                        t = t0 + b
                        gwt(t, b)
                        sst(t, b)
                    for b in range(2):
                        t = t0 + b
                        swt(t, b)
                        gst(t + 2, b)

                for b in range(2):
                    t = GCHA - 2 + b
                    gwt(t, b)
                    sst(t, b)
                for b in range(2):
                    swt(GCHA - 2 + b, b)

            plsc.subcore_barrier()
            rows = ACC_ROWS // NSUB
            pltpu.sync_copy(acc_sp.at[pl.ds(s * rows, rows)],
                            out_slice.at[pl.ds(s * rows, rows)])

        @pl.when(c == 0)
        def _():
            run(ylo_hbm, out_hbm.at[0])

        @pl.when(c == 1)
        def _():
            run(yhi_hbm, out_hbm.at[1])

    return k(y_lo, y_hi, gidx_p, sidx_p, zeros128)


def kernel(node_features, edge_index, edge_type, node_type, params):
    src, dst = edge_index[0], edge_index[1]
    et = edge_type
    pad = EP - 2 * NE
    gidx = jnp.concatenate(
        [et * N + src, (et + NBASE) * N + dst,
         jnp.zeros((pad,), jnp.int32)]).reshape(NSUB, NCH, CHUNK)
    sidx = jnp.concatenate(
        [dst, src,
         jnp.full((pad,), DUMMY, jnp.int32)]).reshape(NSUB, NCH, CHUNK)
    cidx = jnp.concatenate(
        [et * N + dst, (et + NBASE) * N + src,
         jnp.full((pad,), CNT_DUMMY, jnp.int32)])
    clo = ((cidx & 7) + 8 * ((cidx >> 3) & 127)).reshape(
        2 * NSUB, NCH // 2, CHUNK)
    chi = (cidx >> 3).reshape(2 * NSUB, NCH // 2, CHUNK)
    onesrel = (jnp.arange(CHUNK)[None, :] // 16
               == (jnp.arange(SPREAD)[:, None] & 7)).astype(jnp.float32)
    zeros128 = jnp.zeros((CHUNK, H), jnp.float32)

    cgrid = _sc_counts(clo, chi, onesrel, zeros128)

    def _cnt_t(g):
        cv = g.reshape(CNT_VROWS, 8, 16)[:, :, 0].reshape(CNT_ROWS)
        return cv[:NREL * N].reshape(NREL, N).T

    cnt_ta, cnt_tb = _cnt_t(cgrid[0]), _cnt_t(cgrid[1])

    p = params
    r2 = lambda a: a.reshape(1, -1)
    base = _tc_base(node_features, node_type.reshape(N, 1),
                    p["node_type_embed"], r2(p["in_g"]), r2(p["in_b"]))

    x = base
    for lp in p["layers"]:
        y_lo, y_hi = _tc_project(x, lp["rel_W"])
        acc = _sc_agg(y_lo, y_hi, gidx, sidx, zeros128)
        x = _tc_post(x, acc[0, :N], acc[1, :N], cnt_ta, cnt_tb, lp["rel_b"],
                     lp["self_W"], r2(lp["self_b"]),
                     r2(lp["n1_g"]), r2(lp["n1_b"]),
                     lp["ff_W1"], r2(lp["ff_b1"]),
                     lp["ff_W2"], r2(lp["ff_b2"]),
                     r2(lp["n2_g"]), r2(lp["n2_b"]))

    gm = jax.nn.sigmoid(p["graph_mix_logit"])
    dm = jax.nn.sigmoid(p["doc_mix_logit"])
    xf, pooled = _tc_final(x, base, r2(p["out_g"]), r2(p["out_b"]),
                           jnp.full((1, D), gm, jnp.float32),
                           jnp.full((1, D), dm, jnp.float32))
    return xf, pooled.reshape(D)


# speedup vs baseline: 1.1082x; 1.1082x over previous
import functools

import jax
import jax.numpy as jnp
from jax import lax
from jax.experimental import pallas as pl
from jax.experimental.pallas import tpu as pltpu
from jax.experimental.pallas import tpu_sc as plsc

N = 10000
D = 256
H = 128
NBASE = 2
NREL = 4
NE = 160000
EPS = 1e-5

CHUNK = 128
NCH = 160
GCH = 16
GCHA = 32
NSUB = 16
EP = NSUB * NCH * CHUNK
ACC_ROWS = 10240
DUMMY = N
CNT_ROWS = 40960
CNT_DUMMY = NREL * N


def _ln(t, g, b):
    mu = jnp.mean(t, axis=-1, keepdims=True)
    var = jnp.mean((t - mu) ** 2, axis=-1, keepdims=True)
    return (t - mu) * lax.rsqrt(var + EPS) * g + b


def _erf(z):
    a1, a2, a3, a4, a5 = (0.254829592, -0.284496736, 1.421413741,
                          -1.453152027, 1.061405429)
    p = 0.3275911
    az = jnp.abs(z)
    t = 1.0 / (1.0 + p * az)
    poly = ((((a5 * t + a4) * t + a3) * t + a2) * t + a1) * t
    e = 1.0 - poly * jnp.exp(-az * az)
    return jnp.sign(z) * e


def _gelu(u):
    return 0.5 * u * (1.0 + _erf(u * 0.7071067811865476))


def _dot(a, b):
    return lax.dot_general(a, b, (((1,), (0,)), ((), ())),
                           preferred_element_type=jnp.float32)


def _tc_base(nf, nt2, emb, g, b):
    def body(nf_ref, nt_ref, emb_ref, g_ref, b_ref, o_ref):
        nt = nt_ref[...]
        e = jnp.where(nt == 0, emb_ref[0:1, :],
                      jnp.where(nt == 1, emb_ref[1:2, :], emb_ref[2:3, :]))
        o_ref[...] = _ln(nf_ref[...] + e, g_ref[...], b_ref[...])

    R = 1000
    return pl.pallas_call(
        body,
        grid=(N // R,),
        in_specs=[
            pl.BlockSpec((R, D), lambda i: (i, 0)),
            pl.BlockSpec((R, 1), lambda i: (i, 0)),
            pl.BlockSpec((3, D), lambda i: (0, 0)),
            pl.BlockSpec((1, D), lambda i: (0, 0)),
            pl.BlockSpec((1, D), lambda i: (0, 0)),
        ],
        out_specs=pl.BlockSpec((R, D), lambda i: (i, 0)),
        out_shape=jax.ShapeDtypeStruct((N, D), jnp.float32),
    )(nf, nt2, emb, g, b)


def _tc_project(x, rel_w):
    def body(x_ref, w_ref, lo_ref, hi_ref):
        z = _dot(x_ref[...], w_ref[0])
        lo_ref[...] = z[:, :H]
        hi_ref[...] = z[:, H:]

    R = 1000
    nrb = N // R
    return pl.pallas_call(
        body,
        grid=(NREL, nrb),
        in_specs=[
            pl.BlockSpec((R, D), lambda r, i: (i, 0)),
            pl.BlockSpec((1, D, D), lambda r, i: (r, 0, 0)),
        ],
        out_specs=[
            pl.BlockSpec((R, H), lambda r, i: (r * nrb + i, 0)),
            pl.BlockSpec((R, H), lambda r, i: (r * nrb + i, 0)),
        ],
        out_shape=[
            jax.ShapeDtypeStruct((NREL * N, H), jnp.float32),
            jax.ShapeDtypeStruct((NREL * N, H), jnp.float32),
        ],
    )(x, rel_w)


def _tc_post(x, acc_lo, acc_hi, cnt_ta, cnt_tb, ball, sw, sb, n1g, n1b,
             w1, b1, w2, b2, n2g, n2b):
    def body(x_ref, lo_ref, hi_ref, cnta_ref, cntb_ref, ball_ref, sw_ref,
             sb_ref, n1g_ref, n1b_ref, w1_ref, b1_ref, w2_ref, b2_ref,
             n2g_ref, n2b_ref, o_ref):
        cnt = cnta_ref[...] + cntb_ref[...]
        deg = jnp.sum(cnt, axis=1, keepdims=True)
        bias_term = _dot(cnt, ball_ref[...])
        agg_pre = jnp.concatenate([lo_ref[...], hi_ref[...]], axis=1)
        agg = (agg_pre + bias_term) / jnp.maximum(deg, 1.0)
        xv = x_ref[...]
        h = _dot(xv, sw_ref[...]) + sb_ref[...] + agg
        x1 = _ln(xv + h, n1g_ref[...], n1b_ref[...])
        u = _gelu(_dot(x1, w1_ref[...]) + b1_ref[...])
        ff = _dot(u, w2_ref[...]) + b2_ref[...]
        o_ref[...] = _ln(x1 + ff, n2g_ref[...], n2b_ref[...])

    R = 1000
    return pl.pallas_call(
        body,
        grid=(N // R,),
        in_specs=[
            pl.BlockSpec((R, D), lambda i: (i, 0)),
            pl.BlockSpec((R, H), lambda i: (i, 0)),
            pl.BlockSpec((R, H), lambda i: (i, 0)),
            pl.BlockSpec((R, NREL), lambda i: (i, 0)),
            pl.BlockSpec((R, NREL), lambda i: (i, 0)),
            pl.BlockSpec((NREL, D), lambda i: (0, 0)),
            pl.BlockSpec((D, D), lambda i: (0, 0)),
            pl.BlockSpec((1, D), lambda i: (0, 0)),
            pl.BlockSpec((1, D), lambda i: (0, 0)),
            pl.BlockSpec((1, D), lambda i: (0, 0)),
            pl.BlockSpec((D, 2 * D), lambda i: (0, 0)),
            pl.BlockSpec((1, 2 * D), lambda i: (0, 0)),
            pl.BlockSpec((2 * D, D), lambda i: (0, 0)),
            pl.BlockSpec((1, D), lambda i: (0, 0)),
            pl.BlockSpec((1, D), lambda i: (0, 0)),
            pl.BlockSpec((1, D), lambda i: (0, 0)),
        ],
        out_specs=pl.BlockSpec((R, D), lambda i: (i, 0)),
        out_shape=jax.ShapeDtypeStruct((N, D), jnp.float32),
    )(x, acc_lo, acc_hi, cnt_ta, cnt_tb, ball, sw, sb, n1g, n1b,
      w1, b1, w2, b2, n2g, n2b)


def _tc_final(x, base, og, ob, gmv, dmv):
    def body(x_ref, base_ref, og_ref, ob_ref, gm_ref, dm_ref,
             o_ref, pool_ref):
        xln = _ln(x_ref[...], og_ref[...], ob_ref[...])
        bv = base_ref[...]
        xf = bv + gm_ref[...] * (xln - bv)
        o_ref[...] = xf
        i = pl.program_id(0)

        @pl.when(i == 0)
        def _():
            pool_ref[...] = jnp.zeros_like(pool_ref)

        dm = dm_ref[...]
        pool_ref[...] += jnp.sum(xf, axis=0, keepdims=True) * ((1.0 - dm) / N)

        @pl.when(i == 0)
        def _():
            pool_ref[...] += dm * xf[0:1, :]

    R = 1000
    return pl.pallas_call(
        body,
        grid=(N // R,),
        in_specs=[
            pl.BlockSpec((R, D), lambda i: (i, 0)),
            pl.BlockSpec((R, D), lambda i: (i, 0)),
            pl.BlockSpec((1, D), lambda i: (0, 0)),
            pl.BlockSpec((1, D), lambda i: (0, 0)),
            pl.BlockSpec((1, D), lambda i: (0, 0)),
            pl.BlockSpec((1, D), lambda i: (0, 0)),
        ],
        out_specs=[
            pl.BlockSpec((R, D), lambda i: (i, 0)),
            pl.BlockSpec((1, D), lambda i: (0, 0)),
        ],
        out_shape=[
            jax.ShapeDtypeStruct((N, D), jnp.float32),
            jax.ShapeDtypeStruct((1, D), jnp.float32),
        ],
    )(x, base, og, ob, gmv, dmv)


CNT_VROWS = CNT_ROWS // 8


SPREAD = 1024


def _sc_counts(clo_p, chi_p, onesrel, zeros128):
    mesh = plsc.VectorSubcoreMesh(core_axis_name="c", subcore_axis_name="s")

    @functools.partial(
        pl.kernel,
        mesh=mesh,
        out_type=jax.ShapeDtypeStruct((2, CNT_VROWS, CHUNK), jnp.float32),
        scratch_types=[
            pltpu.VMEM((GCH, CHUNK), jnp.int32),
            pltpu.VMEM((GCH, CHUNK), jnp.int32),
            pltpu.VMEM((2, CHUNK, CHUNK), jnp.float32),
            pltpu.VMEM_SHARED((CNT_VROWS, CHUNK), jnp.float32),
            pltpu.SemaphoreType.DMA((2,)),
            pltpu.SemaphoreType.DMA,
        ],
    )
    def k(clo_hbm, chi_hbm, ones_hbm, zer_hbm, out_hbm,
          lo_v, hi_v, ring_v, cnt_sp, gsem, sem):
        c = lax.axis_index("c")
        s = lax.axis_index("s")
        w = c * NSUB + s
        pltpu.sync_copy(zer_hbm, ring_v.at[0])
        rows = CNT_VROWS // NSUB
        zb = s * rows
        for kk in range(2):
            pltpu.sync_copy(ring_v.at[0],
                            cnt_sp.at[pl.ds(zb + kk * CHUNK, CHUNK)])
        pltpu.sync_copy(ring_v.at[0].at[pl.ds(0, 64)],
                        cnt_sp.at[pl.ds(zb + 2 * CHUNK, 64)])
        plsc.subcore_barrier()

        def gst(t, b):
            pltpu.make_async_copy(ones_hbm.at[lo_v.at[t]], ring_v.at[b],
                                  gsem.at[b]).start()

        def gwt(t, b):
            pltpu.make_async_copy(ones_hbm.at[lo_v.at[t]], ring_v.at[b],
                                  gsem.at[b]).wait()

        def sct(t, b):
            pltpu.sync_copy(ring_v.at[b], cnt_sp.at[hi_v.at[t]], add=True)

        @pl.loop(0, NCH // 2, step=GCH)
        def _(j0):
            pltpu.async_copy(clo_hbm.at[w, pl.ds(j0, GCH)], lo_v, sem).wait()
            pltpu.async_copy(chi_hbm.at[w, pl.ds(j0, GCH)], hi_v, sem).wait()
            gst(0, 0)
            gst(1, 1)

            @pl.loop(0, GCH - 2, step=2)
            def _(t0):
                for b in range(2):
                    t = t0 + b
                    gwt(t, b)
                    sct(t, b)
                    gst(t + 2, b)

            for b in range(2):
                t = GCH - 2 + b
                gwt(t, b)
                sct(t, b)

        plsc.subcore_barrier()

        @pl.when(c == 0)
        def _():
            pltpu.sync_copy(cnt_sp.at[pl.ds(zb, rows)],
                            out_hbm.at[0].at[pl.ds(zb, rows)])

        @pl.when(c == 1)
        def _():
            pltpu.sync_copy(cnt_sp.at[pl.ds(zb, rows)],
                            out_hbm.at[1].at[pl.ds(zb, rows)])

    return k(clo_p, chi_p, onesrel, zeros128)


def _sc_agg(y_lo, y_hi, gidx_p, sidx_p, zeros128):
    mesh = plsc.VectorSubcoreMesh(core_axis_name="c", subcore_axis_name="s")

    @functools.partial(
        pl.kernel,
        mesh=mesh,
        out_type=jax.ShapeDtypeStruct((2, ACC_ROWS, H), jnp.float32),
        scratch_types=[
            pltpu.VMEM((GCHA, CHUNK), jnp.int32),
            pltpu.VMEM((GCHA, CHUNK), jnp.int32),
            pltpu.VMEM((2, CHUNK, H), jnp.float32),
            pltpu.VMEM_SHARED((ACC_ROWS, H), jnp.float32),
            pltpu.SemaphoreType.DMA((2,)),
            pltpu.SemaphoreType.DMA((2,)),
            pltpu.SemaphoreType.DMA,
        ],
    )
    def k(ylo_hbm, yhi_hbm, g_hbm, s_hbm, z_hbm, out_hbm,
          g_v, s_v, ring_v, acc_sp, gsem, ssem, sem):
        c = lax.axis_index("c")
        s = lax.axis_index("s")
        pltpu.sync_copy(z_hbm, ring_v.at[0])
        zb = s * (ACC_ROWS // NSUB)
        for kk in range(ACC_ROWS // NSUB // CHUNK):
            pltpu.sync_copy(ring_v.at[0], acc_sp.at[pl.ds(zb + kk * CHUNK,
                                                          CHUNK)])
        plsc.subcore_barrier()

        def run(y_hbm, out_slice):
            def gst(t, b):
                pltpu.make_async_copy(y_hbm.at[g_v.at[t]], ring_v.at[b],
                                      gsem.at[b]).start()

            def gwt(t, b):
                pltpu.make_async_copy(y_hbm.at[g_v.at[t]], ring_v.at[b],
                                      gsem.at[b]).wait()

            def sct(t, b):
                pltpu.sync_copy(ring_v.at[b], acc_sp.at[s_v.at[t]],
                                add=True)

            @pl.loop(0, NCH, step=GCHA)
            def _(j0):
                pltpu.async_copy(g_hbm.at[s, pl.ds(j0, GCHA)], g_v,
                                 sem).wait()
                pltpu.async_copy(s_hbm.at[s, pl.ds(j0, GCHA)], s_v,
                                 sem).wait()
                gst(0, 0)
                gst(1, 1)

                @pl.loop(0, GCHA - 2, step=2)
                def _(t0):
                    for b in range(2):
                        t = t0 + b
                        gwt(t, b)
                        sct(t, b)
                        gst(t + 2, b)

                for b in range(2):
                    t = GCHA - 2 + b
                    gwt(t, b)
                    sct(t, b)

            plsc.subcore_barrier()
            rows = ACC_ROWS // NSUB
            pltpu.sync_copy(acc_sp.at[pl.ds(s * rows, rows)],
                            out_slice.at[pl.ds(s * rows, rows)])

        @pl.when(c == 0)
        def _():
            run(ylo_hbm, out_hbm.at[0])

        @pl.when(c == 1)
        def _():
            run(yhi_hbm, out_hbm.at[1])

    return k(y_lo, y_hi, gidx_p, sidx_p, zeros128)


def kernel(node_features, edge_index, edge_type, node_type, params):
    src, dst = edge_index[0], edge_index[1]
    et = edge_type
    pad = EP - 2 * NE
    gidx = jnp.concatenate(
        [et * N + src, (et + NBASE) * N + dst,
         jnp.zeros((pad,), jnp.int32)]).reshape(NSUB, NCH, CHUNK)
    sidx = jnp.concatenate(
        [dst, src,
         jnp.full((pad,), DUMMY, jnp.int32)]).reshape(NSUB, NCH, CHUNK)
    cidx = jnp.concatenate(
        [et * N + dst, (et + NBASE) * N + src,
         jnp.full((pad,), CNT_DUMMY, jnp.int32)])
    clo = ((cidx & 7) + 8 * ((cidx >> 3) & 127)).reshape(
        2 * NSUB, NCH // 2, CHUNK)
    chi = (cidx >> 3).reshape(2 * NSUB, NCH // 2, CHUNK)
    onesrel = (jnp.arange(CHUNK)[None, :] // 16
               == (jnp.arange(SPREAD)[:, None] & 7)).astype(jnp.float32)
    zeros128 = jnp.zeros((CHUNK, H), jnp.float32)

    cgrid = _sc_counts(clo, chi, onesrel, zeros128)

    def _cnt_t(g):
        cv = g.reshape(CNT_VROWS, 8, 16)[:, :, 0].reshape(CNT_ROWS)
        return cv[:NREL * N].reshape(NREL, N).T

    cnt_ta, cnt_tb = _cnt_t(cgrid[0]), _cnt_t(cgrid[1])

    p = params
    r2 = lambda a: a.reshape(1, -1)
    base = _tc_base(node_features, node_type.reshape(N, 1),
                    p["node_type_embed"], r2(p["in_g"]), r2(p["in_b"]))

    x = base
    for lp in p["layers"]:
        y_lo, y_hi = _tc_project(x, lp["rel_W"])
        acc = _sc_agg(y_lo, y_hi, gidx, sidx, zeros128)
        x = _tc_post(x, acc[0, :N], acc[1, :N], cnt_ta, cnt_tb, lp["rel_b"],
                     lp["self_W"], r2(lp["self_b"]),
                     r2(lp["n1_g"]), r2(lp["n1_b"]),
                     lp["ff_W1"], r2(lp["ff_b1"]),
                     lp["ff_W2"], r2(lp["ff_b2"]),
                     r2(lp["n2_g"]), r2(lp["n2_b"]))

    gm = jax.nn.sigmoid(p["graph_mix_logit"])
    dm = jax.nn.sigmoid(p["doc_mix_logit"])
    xf, pooled = _tc_final(x, base, r2(p["out_g"]), r2(p["out_b"]),
                           jnp.full((1, D), gm, jnp.float32),
                           jnp.full((1, D), dm, jnp.float32))
    return xf, pooled.reshape(D)
